# baseline (speedup 1.0000x reference)
import jax
import jax.numpy as jnp
from jax.experimental import pallas as pl
from jax.experimental.pallas import tpu as pltpu

_N = 2048
_K = 16
_BK = _N // _K

_DN_T = (((0,), (0,)), ((), ()))
_DN_R = (((1,), (1,)), ((), ()))


def _gnn_fused(adj_ref, x_ref, w1r_ref, w1s_ref, b1_ref, w2r_ref, w2s_ref,
               b2_ref, out_ref, af_scr, acc_scr):
    i = pl.program_id(0)
    ab = adj_ref[...].astype(jnp.float32)
    af_scr[pl.ds(i * _BK, _BK), :] = ab
    xb = x_ref[pl.ds(i * _BK, _BK), :]
    part = jax.lax.dot_general(ab, xb, _DN_T,
                               preferred_element_type=jnp.float32)

    @pl.when(i == 0)
    def _init():
        acc_scr[...] = part

    @pl.when(i > 0)
    def _accum():
        acc_scr[...] += part

    @pl.when(i == _K - 1)
    def _finish():
        x = x_ref[...]
        h = (jax.lax.dot_general(acc_scr[...], w1r_ref[...], _DN_R,
                                 preferred_element_type=jnp.float32)
             + b1_ref[...]
             + jax.lax.dot_general(x, w1s_ref[...], _DN_R,
                                   preferred_element_type=jnp.float32))
        h = jnp.maximum(h, 0.0)
        h2 = jax.lax.dot_general(h, w2r_ref[...], _DN_R,
                                 preferred_element_type=jnp.float32)
        out = (jax.lax.dot_general(af_scr[...], h2, _DN_T,
                                   preferred_element_type=jnp.float32)
               + b2_ref[...]
               + jax.lax.dot_general(h, w2s_ref[...], _DN_R,
                                     preferred_element_type=jnp.float32))
        shifted = out - jnp.max(out, axis=1, keepdims=True)
        out_ref[...] = shifted - jnp.log(
            jnp.sum(jnp.exp(shifted), axis=1, keepdims=True))


def kernel(x, adj, W1_rel, b1_rel, W1_root, W2_rel, b2_rel, W2_root):
    in_ch = x.shape[1]
    hid_ch = W1_rel.shape[0]
    out_ch = W2_rel.shape[0]
    return pl.pallas_call(
        _gnn_fused,
        grid=(_K,),
        in_specs=[
            pl.BlockSpec((_BK, _N), lambda i: (i, 0)),
            pl.BlockSpec((_N, in_ch), lambda i: (0, 0)),
            pl.BlockSpec((hid_ch, in_ch), lambda i: (0, 0)),
            pl.BlockSpec((hid_ch, in_ch), lambda i: (0, 0)),
            pl.BlockSpec((1, hid_ch), lambda i: (0, 0)),
            pl.BlockSpec((out_ch, hid_ch), lambda i: (0, 0)),
            pl.BlockSpec((out_ch, hid_ch), lambda i: (0, 0)),
            pl.BlockSpec((1, out_ch), lambda i: (0, 0)),
        ],
        out_specs=pl.BlockSpec((_N, out_ch), lambda i: (0, 0)),
        out_shape=jax.ShapeDtypeStruct((_N, out_ch), jnp.float32),
        scratch_shapes=[
            pltpu.VMEM((_N, _N), jnp.float32),
            pltpu.VMEM((_N, hid_ch), jnp.float32),
        ],
    )(adj, x, W1_rel, W1_root, b1_rel.reshape(1, -1),
      W2_rel, W2_root, b2_rel.reshape(1, -1))


# speedup vs baseline: 1.1772x; 1.1772x over previous
import jax
import jax.numpy as jnp
from jax.experimental import pallas as pl

_N = 2048

_DN_T = (((0,), (0,)), ((), ()))
_DN_R = (((1,), (1,)), ((), ()))


def _gnn_fused(adj_ref, x_ref, w1r_ref, w1s_ref, b1_ref, w2r_ref, w2s_ref,
               b2_ref, out_ref):
    a = adj_ref[...].astype(jnp.float32)
    x = x_ref[...]
    aggr1 = jax.lax.dot_general(a, x, _DN_T, preferred_element_type=jnp.float32)
    h = (jax.lax.dot_general(aggr1, w1r_ref[...], _DN_R,
                             preferred_element_type=jnp.float32)
         + b1_ref[...]
         + jax.lax.dot_general(x, w1s_ref[...], _DN_R,
                               preferred_element_type=jnp.float32))
    h = jnp.maximum(h, 0.0)
    h2 = jax.lax.dot_general(h, w2r_ref[...], _DN_R,
                             preferred_element_type=jnp.float32)
    out = (jax.lax.dot_general(a, h2, _DN_T, preferred_element_type=jnp.float32)
           + b2_ref[...]
           + jax.lax.dot_general(h, w2s_ref[...], _DN_R,
                                 preferred_element_type=jnp.float32))
    shifted = out - jnp.max(out, axis=1, keepdims=True)
    out_ref[...] = shifted - jnp.log(
        jnp.sum(jnp.exp(shifted), axis=1, keepdims=True))


def kernel(x, adj, W1_rel, b1_rel, W1_root, W2_rel, b2_rel, W2_root):
    out_ch = W2_rel.shape[0]
    return pl.pallas_call(
        _gnn_fused,
        out_shape=jax.ShapeDtypeStruct((_N, out_ch), jnp.float32),
    )(adj, x, W1_rel, W1_root, b1_rel.reshape(1, -1),
      W2_rel, W2_root, b2_rel.reshape(1, -1))
